# baseline (speedup 1.0000x reference)
import jax
import jax.numpy as jnp
from jax.experimental import pallas as pl
from jax.experimental.pallas import tpu as pltpu

_PREC = jax.lax.Precision.HIGHEST

_B, _N, _F, _H, _C = 8, 1024, 512, 256, 16
_BB = 2


def _fused_body(x_ref, maskf_ref, g_ref, w1x_ref, w1c_ref, b1_ref, w2_ref,
                wc_ref, bc_ref, wih_ref, whh_ref, bih_ref, bhh_ref,
                cf_ref, asn_ref):
    f32 = jnp.float32
    x = x_ref[...]
    x2 = x.reshape(_BB * _N, _F)

    xw = jax.lax.dot_general(x2, w1x_ref[...], (((1,), (1,)), ((), ())),
                             preferred_element_type=f32, precision=_PREC)
    xw = xw.reshape(_BB, _N, _H)

    xm = jnp.mean(x, axis=1)
    gc = jax.lax.dot_general(xm, wc_ref[...], (((1,), (1,)), ((), ())),
                             preferred_element_type=f32, precision=_PREC) + bc_ref[...]

    avail = maskf_ref[:, 0, :] > 0.5
    h = jnp.zeros((_BB, _H), f32)
    iota_n = jax.lax.broadcasted_iota(jnp.int32, (_BB, _N), 1)
    b1 = b1_ref[...]
    w2 = w2_ref[...]
    wih = wih_ref[...]
    whh = whh_ref[...]
    bih = bih_ref[...]
    bhh = bhh_ref[...]

    gis = []
    onehots = []
    for c in range(_C):
        d = jax.lax.dot_general(gc, w1c_ref[...], (((1,), (1,)), ((), ())),
                                preferred_element_type=f32, precision=_PREC) + b1
        t = jnp.maximum(xw + d[:, None, :], 0.0)
        logits = jax.lax.dot_general(t.reshape(_BB * _N, _H), w2,
                                     (((1,), (1,)), ((), ())),
                                     preferred_element_type=f32, precision=_PREC)
        logits = logits.reshape(_BB, _N)
        noisy = jnp.where(avail, logits + g_ref[:, c, :], -jnp.inf)

        m = jnp.max(noisy, axis=-1, keepdims=True)
        idx = jnp.min(jnp.where(noisy == m, iota_n, _N),
                      axis=-1, keepdims=True)
        has = jnp.any(avail, axis=-1, keepdims=True)
        sel = (iota_n == idx) & has
        onehot = sel.astype(f32)
        onehots.append(onehot)

        emb = jax.lax.dot_general(onehot, x, (((1,), (1,)), ((0,), (0,))),
                                  preferred_element_type=f32, precision=_PREC)
        cf_ref[:, c, :] = emb

        gi = jax.lax.dot_general(emb, wih, (((1,), (1,)), ((), ())),
                                 preferred_element_type=f32, precision=_PREC) + bih
        gis.append(gi)
        for t_ in range(c + 1):
            git = gis[t_]
            gh = jax.lax.dot_general(h, whh, (((1,), (1,)), ((), ())),
                                     preferred_element_type=f32, precision=_PREC) + bhh
            r = jax.nn.sigmoid(git[:, 0:_H] + gh[:, 0:_H])
            z = jax.nn.sigmoid(git[:, _H:2 * _H] + gh[:, _H:2 * _H])
            n = jnp.tanh(git[:, 2 * _H:] + r * gh[:, 2 * _H:])
            h = (1.0 - z) * n + z * h
        gc = h
        avail = avail & (~sel)

    for c in range(_C):
        asn_ref[:, c, :] = onehots[c]


def kernel(x, adj, mask, W1, b1, W2, b2, Wc, bc, W_ih, W_hh, b_ih, b_hh):
    del adj, b2
    f32 = jnp.float32

    noise_key = jax.random.key(123)
    g = jnp.stack([
        -jnp.log(-jnp.log(
            jax.random.uniform(jax.random.fold_in(noise_key, c), (_B, _N),
                               dtype=f32) + 1e-8) + 1e-8)
        for c in range(_C)
    ])
    g = g.transpose(1, 0, 2)

    maskf = mask.astype(f32).reshape(_B, 1, _N)
    w1x = W1[:, :_F]
    w1c = W1[:, _F:]

    grid = (_B // _BB,)
    full = lambda *shape: pl.BlockSpec(shape, lambda i: (0,) * len(shape))
    cf, asn = pl.pallas_call(
        _fused_body,
        grid=grid,
        in_specs=[
            pl.BlockSpec((_BB, _N, _F), lambda i: (i, 0, 0)),
            pl.BlockSpec((_BB, 1, _N), lambda i: (i, 0, 0)),
            pl.BlockSpec((_BB, _C, _N), lambda i: (i, 0, 0)),
            full(_H, _F),
            full(_H, _H),
            full(1, _H),
            full(1, _H),
            full(_H, _F),
            full(1, _H),
            full(3 * _H, _F),
            full(3 * _H, _H),
            full(1, 3 * _H),
            full(1, 3 * _H),
        ],
        out_specs=[
            pl.BlockSpec((_BB, _C, _F), lambda i: (i, 0, 0)),
            pl.BlockSpec((_BB, _C, _N), lambda i: (i, 0, 0)),
        ],
        out_shape=[
            jax.ShapeDtypeStruct((_B, _C, _F), f32),
            jax.ShapeDtypeStruct((_B, _C, _N), f32),
        ],
    )(x, maskf, g, w1x, w1c, b1.reshape(1, _H), W2,
      Wc, bc.reshape(1, _H), W_ih, W_hh,
      b_ih.reshape(1, 3 * _H), b_hh.reshape(1, 3 * _H))

    asn = asn.transpose(0, 2, 1)
    cluster_adj = jnp.broadcast_to(
        (jnp.ones((_C, _C), f32) - jnp.eye(_C, dtype=f32))[None], (_B, _C, _C))
    return cf, cluster_adj, asn


# speedup vs baseline: 2.3788x; 2.3788x over previous
import jax
import jax.numpy as jnp
from jax.experimental import pallas as pl
from jax.experimental.pallas import tpu as pltpu

_PREC = jax.lax.Precision.HIGHEST

_B, _N, _F, _H, _C = 8, 1024, 512, 256, 16
_NC = 256
_NCH = _N // _NC


def _dot_nt(a, b):
    return jax.lax.dot_general(a, b, (((1,), (1,)), ((), ())),
                               preferred_element_type=jnp.float32,
                               precision=_PREC)


def _fused_body(x_ref, maskf_ref, g_ref, w1x_ref, w1c_ref, b1_ref, w2_ref,
                wc_ref, bc_ref, wih_ref, whh_ref, bih_ref, bhh_ref,
                cf_ref, asn_ref, xw_ref):
    f32 = jnp.float32
    w1x = w1x_ref[...]

    xsum = jnp.zeros((_B, _F), f32)
    for j in range(_NCH):
        xc = x_ref[:, j * _NC:(j + 1) * _NC, :]
        xw_ref[:, j * _NC:(j + 1) * _NC, :] = _dot_nt(
            xc.reshape(_B * _NC, _F), w1x).reshape(_B, _NC, _H)
        xsum = xsum + jnp.sum(xc, axis=1)
    xm = xsum * (1.0 / _N)

    gc = _dot_nt(xm, wc_ref[...]) + bc_ref[...]

    avail = maskf_ref[:, 0, :] > 0.5
    h = jnp.zeros((_B, _H), f32)
    iota_n = jax.lax.broadcasted_iota(jnp.int32, (_B, _N), 1)
    b1 = b1_ref[...]
    w2 = w2_ref[...]
    wih = wih_ref[...]
    whh = whh_ref[...]
    bih = bih_ref[...]
    bhh = bhh_ref[...]

    gis = []
    for c in range(_C):
        d = _dot_nt(gc, w1c_ref[...]) + b1
        lparts = []
        for j in range(_NCH):
            t = jnp.maximum(xw_ref[:, j * _NC:(j + 1) * _NC, :]
                            + d[:, None, :], 0.0)
            lg = _dot_nt(t.reshape(_B * _NC, _H), w2)
            lparts.append(lg.reshape(_B, _NC))
        logits = jnp.concatenate(lparts, axis=1)
        noisy = jnp.where(avail, logits + g_ref[:, c, :], -jnp.inf)

        m = jnp.max(noisy, axis=-1, keepdims=True)
        idx = jnp.min(jnp.where(noisy == m, iota_n, _N),
                      axis=-1, keepdims=True)
        has = jnp.any(avail, axis=-1, keepdims=True)
        sel = (iota_n == idx) & has
        onehot = sel.astype(f32)
        asn_ref[:, c, :] = onehot

        emb = jnp.zeros((_B, _F), f32)
        for j in range(_NCH):
            emb = emb + jax.lax.dot_general(
                onehot[:, j * _NC:(j + 1) * _NC],
                x_ref[:, j * _NC:(j + 1) * _NC, :],
                (((1,), (1,)), ((0,), (0,))),
                preferred_element_type=f32, precision=_PREC)
        cf_ref[:, c, :] = emb

        gi = _dot_nt(emb, wih) + bih
        gis.append(gi)
        for t_ in range(c + 1):
            git = gis[t_]
            gh = _dot_nt(h, whh) + bhh
            r = jax.nn.sigmoid(git[:, 0:_H] + gh[:, 0:_H])
            z = jax.nn.sigmoid(git[:, _H:2 * _H] + gh[:, _H:2 * _H])
            n = jnp.tanh(git[:, 2 * _H:] + r * gh[:, 2 * _H:])
            h = (1.0 - z) * n + z * h
        gc = h
        avail = avail & (~sel)


def kernel(x, adj, mask, W1, b1, W2, b2, Wc, bc, W_ih, W_hh, b_ih, b_hh):
    del adj, b2
    f32 = jnp.float32

    noise_key = jax.random.key(123)
    g = jnp.stack([
        -jnp.log(-jnp.log(
            jax.random.uniform(jax.random.fold_in(noise_key, c), (_B, _N),
                               dtype=f32) + 1e-8) + 1e-8)
        for c in range(_C)
    ])
    g = g.transpose(1, 0, 2)

    maskf = mask.astype(f32).reshape(_B, 1, _N)
    w1x = W1[:, :_F]
    w1c = W1[:, _F:]

    cf, asn = pl.pallas_call(
        _fused_body,
        out_shape=[
            jax.ShapeDtypeStruct((_B, _C, _F), f32),
            jax.ShapeDtypeStruct((_B, _C, _N), f32),
        ],
        scratch_shapes=[pltpu.VMEM((_B, _N, _H), f32)],
    )(x, maskf, g, w1x, w1c, b1.reshape(1, _H), W2,
      Wc, bc.reshape(1, _H), W_ih, W_hh,
      b_ih.reshape(1, 3 * _H), b_hh.reshape(1, 3 * _H))

    asn = asn.transpose(0, 2, 1)
    cluster_adj = jnp.broadcast_to(
        (jnp.ones((_C, _C), f32) - jnp.eye(_C, dtype=f32))[None], (_B, _C, _C))
    return cf, cluster_adj, asn


# speedup vs baseline: 2.4257x; 1.0197x over previous
import jax
import jax.numpy as jnp
from jax.experimental import pallas as pl
from jax.experimental.pallas import tpu as pltpu

_PREC = jax.lax.Precision.HIGHEST
_PREC3 = jax.lax.Precision.HIGHEST

_B, _N, _F, _H, _C = 8, 1024, 512, 256, 16
_NC = 256
_NCH = _N // _NC


def _dot_nt(a, b, prec=_PREC3):
    return jax.lax.dot_general(a, b, (((1,), (1,)), ((), ())),
                               preferred_element_type=jnp.float32,
                               precision=prec)


def _fused_body(x_ref, maskf_ref, g_ref, w1x_ref, w1c_ref, b1_ref, w2_ref,
                wc_ref, bc_ref, wih_ref, whh_ref, bih_ref, bhh_ref,
                cf_ref, asn_ref, xw_ref):
    f32 = jnp.float32
    w1x = w1x_ref[...]

    xsum = jnp.zeros((_B, _F), f32)
    for j in range(_NCH):
        xc = x_ref[:, j * _NC:(j + 1) * _NC, :]
        xw_ref[:, j * _NC:(j + 1) * _NC, :] = _dot_nt(
            xc.reshape(_B * _NC, _F), w1x).reshape(_B, _NC, _H)
        xsum = xsum + jnp.sum(xc, axis=1)
    xm = xsum * (1.0 / _N)

    gc = _dot_nt(xm, wc_ref[...]) + bc_ref[...]

    avail = maskf_ref[:, 0, :] > 0.5
    h = jnp.zeros((_B, _H), f32)
    iota_n = jax.lax.broadcasted_iota(jnp.int32, (_B, _N), 1)
    b1 = b1_ref[...]
    w2 = w2_ref[...]
    wih = wih_ref[...]
    whh = whh_ref[...]
    bih = bih_ref[...]
    bhh = bhh_ref[...]

    def gru_step(hh, git):
        gh = _dot_nt(hh, whh) + bhh
        r = jax.nn.sigmoid(git[:, 0:_H] + gh[:, 0:_H])
        z = jax.nn.sigmoid(git[:, _H:2 * _H] + gh[:, _H:2 * _H])
        n = jnp.tanh(git[:, 2 * _H:] + r * gh[:, 2 * _H:])
        return (1.0 - z) * n + z * hh

    gis = []
    for c in range(_C):
        h_pre = h
        for t_ in range(c):
            h_pre = gru_step(h_pre, gis[t_])

        d = _dot_nt(gc, w1c_ref[...]) + b1
        lparts = []
        for j in range(_NCH):
            t = jnp.maximum(xw_ref[:, j * _NC:(j + 1) * _NC, :]
                            + d[:, None, :], 0.0)
            lg = _dot_nt(t.reshape(_B * _NC, _H), w2)
            lparts.append(lg.reshape(_B, _NC))
        logits = jnp.concatenate(lparts, axis=1)
        noisy = jnp.where(avail, logits + g_ref[:, c, :], -jnp.inf)

        m = jnp.max(noisy, axis=-1, keepdims=True)
        idx = jnp.min(jnp.where(noisy == m, iota_n, _N),
                      axis=-1, keepdims=True)
        has = jnp.any(avail, axis=-1, keepdims=True)
        sel = (iota_n == idx) & has
        onehot = sel.astype(f32)
        asn_ref[:, c, :] = onehot

        emb = jnp.zeros((_B, _F), f32)
        for j in range(_NCH):
            emb = emb + jax.lax.dot_general(
                onehot[:, j * _NC:(j + 1) * _NC],
                x_ref[:, j * _NC:(j + 1) * _NC, :],
                (((1,), (1,)), ((0,), (0,))),
                preferred_element_type=f32, precision=_PREC)
        cf_ref[:, c, :] = emb

        gi = _dot_nt(emb, wih) + bih
        gis.append(gi)
        h = gru_step(h_pre, gi)
        gc = h
        avail = avail & (~sel)


def kernel(x, adj, mask, W1, b1, W2, b2, Wc, bc, W_ih, W_hh, b_ih, b_hh):
    del adj, b2
    f32 = jnp.float32

    noise_key = jax.random.key(123)
    g = jnp.stack([
        -jnp.log(-jnp.log(
            jax.random.uniform(jax.random.fold_in(noise_key, c), (_B, _N),
                               dtype=f32) + 1e-8) + 1e-8)
        for c in range(_C)
    ])
    g = g.transpose(1, 0, 2)

    maskf = mask.astype(f32).reshape(_B, 1, _N)
    w1x = W1[:, :_F]
    w1c = W1[:, _F:]

    cf, asn = pl.pallas_call(
        _fused_body,
        out_shape=[
            jax.ShapeDtypeStruct((_B, _C, _F), f32),
            jax.ShapeDtypeStruct((_B, _C, _N), f32),
        ],
        scratch_shapes=[pltpu.VMEM((_B, _N, _H), f32)],
    )(x, maskf, g, w1x, w1c, b1.reshape(1, _H), W2,
      Wc, bc.reshape(1, _H), W_ih, W_hh,
      b_ih.reshape(1, 3 * _H), b_hh.reshape(1, 3 * _H))

    asn = asn.transpose(0, 2, 1)
    cluster_adj = jnp.broadcast_to(
        (jnp.ones((_C, _C), f32) - jnp.eye(_C, dtype=f32))[None], (_B, _C, _C))
    return cf, cluster_adj, asn


# speedup vs baseline: 3.5727x; 1.4728x over previous
import jax
import jax.numpy as jnp
from jax.experimental import pallas as pl
from jax.experimental.pallas import tpu as pltpu

_PREC = jax.lax.Precision.HIGHEST
_PREC3 = jax.lax.Precision.HIGHEST

_B, _N, _F, _H, _C = 8, 1024, 512, 256, 16
_NC = 256
_NCH = _N // _NC


def _dot_nt(a, b, prec=_PREC3):
    return jax.lax.dot_general(a, b, (((1,), (1,)), ((), ())),
                               preferred_element_type=jnp.float32,
                               precision=prec)


def _fused_body(x_ref, maskf_ref, g_ref, w1x_ref, w1c_ref, b1_ref, w2_ref,
                wc_ref, bc_ref, wih_ref, whh_ref, bih_ref, bhh_ref,
                cf_ref, asn_ref, xw_ref):
    f32 = jnp.float32
    w1x = w1x_ref[...]

    xsum = jnp.zeros((_B, _F), f32)
    for j in range(_NCH):
        xc = x_ref[:, j * _NC:(j + 1) * _NC, :]
        xw_ref[:, j * _NC:(j + 1) * _NC, :] = _dot_nt(
            xc.reshape(_B * _NC, _F), w1x).reshape(_B, _NC, _H)
        xsum = xsum + jnp.sum(xc, axis=1)
    xm = xsum * (1.0 / _N)

    gc = _dot_nt(xm, wc_ref[...]) + bc_ref[...]

    avail = maskf_ref[:, 0, :] > 0.5
    h = jnp.zeros((_B, _H), f32)
    iota_n = jax.lax.broadcasted_iota(jnp.int32, (_B, _N), 1)
    b1 = b1_ref[...]
    w2 = w2_ref[...]
    wih = wih_ref[...]
    whh = whh_ref[...]
    bih = bih_ref[...]
    bhh = bhh_ref[...]

    def gru_step(hh, git):
        gh = _dot_nt(hh, whh) + bhh
        r = jax.nn.sigmoid(git[:, 0:_H] + gh[:, 0:_H])
        z = jax.nn.sigmoid(git[:, _H:2 * _H] + gh[:, _H:2 * _H])
        n = jnp.tanh(git[:, 2 * _H:] + r * gh[:, 2 * _H:])
        return (1.0 - z) * n + z * hh

    gis = []
    for c in range(_C):
        h_pre = h
        for t_ in range(c):
            h_pre = gru_step(h_pre, gis[t_])

        d = _dot_nt(gc, w1c_ref[...]) + b1
        lparts = []
        for j in range(_NCH):
            t = jnp.maximum(xw_ref[:, j * _NC:(j + 1) * _NC, :]
                            + d[:, None, :], 0.0)
            lg = _dot_nt(t.reshape(_B * _NC, _H), w2)
            lparts.append(lg.reshape(_B, _NC))
        logits = jnp.concatenate(lparts, axis=1)
        noisy = jnp.where(avail, logits + g_ref[:, c, :], -jnp.inf)

        m = jnp.max(noisy, axis=-1, keepdims=True)
        idx = jnp.min(jnp.where(noisy == m, iota_n, _N),
                      axis=-1, keepdims=True)
        has = jnp.any(avail, axis=-1, keepdims=True)
        sel = (iota_n == idx) & has
        onehot = sel.astype(f32)
        asn_ref[:, c, :] = onehot

        rows = []
        for b in range(_B):
            idx_b = jnp.min(jnp.where(noisy[b:b + 1, :] == m[b:b + 1, :],
                                      iota_n[b:b + 1, :], _N))
            rows.append(x_ref[b, pl.ds(idx_b, 1), :])
        emb = jnp.concatenate(rows, axis=0) * has.astype(f32)
        cf_ref[:, c, :] = emb

        gi = _dot_nt(emb, wih) + bih
        gis.append(gi)
        h = gru_step(h_pre, gi)
        gc = h
        avail = avail & (~sel)


def kernel(x, adj, mask, W1, b1, W2, b2, Wc, bc, W_ih, W_hh, b_ih, b_hh):
    del adj, b2
    f32 = jnp.float32

    noise_key = jax.random.key(123)
    g = jnp.stack([
        -jnp.log(-jnp.log(
            jax.random.uniform(jax.random.fold_in(noise_key, c), (_B, _N),
                               dtype=f32) + 1e-8) + 1e-8)
        for c in range(_C)
    ])
    g = g.transpose(1, 0, 2)

    maskf = mask.astype(f32).reshape(_B, 1, _N)
    w1x = W1[:, :_F]
    w1c = W1[:, _F:]

    cf, asn = pl.pallas_call(
        _fused_body,
        out_shape=[
            jax.ShapeDtypeStruct((_B, _C, _F), f32),
            jax.ShapeDtypeStruct((_B, _C, _N), f32),
        ],
        scratch_shapes=[pltpu.VMEM((_B, _N, _H), f32)],
    )(x, maskf, g, w1x, w1c, b1.reshape(1, _H), W2,
      Wc, bc.reshape(1, _H), W_ih, W_hh,
      b_ih.reshape(1, 3 * _H), b_hh.reshape(1, 3 * _H))

    asn = asn.transpose(0, 2, 1)
    cluster_adj = jnp.broadcast_to(
        (jnp.ones((_C, _C), f32) - jnp.eye(_C, dtype=f32))[None], (_B, _C, _C))
    return cf, cluster_adj, asn


# speedup vs baseline: 3.5838x; 1.0031x over previous
import jax
import jax.numpy as jnp
from jax.experimental import pallas as pl
from jax.experimental.pallas import tpu as pltpu

_PREC = jax.lax.Precision.HIGHEST
_PREC3 = jax.lax.Precision.HIGHEST

_B, _N, _F, _H, _C = 8, 1024, 512, 256, 16
_NC = 256
_NCH = _N // _NC


def _dot_nt(a, b, prec=_PREC3):
    return jax.lax.dot_general(a, b, (((1,), (1,)), ((), ())),
                               preferred_element_type=jnp.float32,
                               precision=prec)


def _fused_body(x_ref, maskf_ref, g_ref, w1x_ref, w1c_ref, b1_ref, w2_ref,
                wc_ref, bc_ref, wih_ref, whh_ref, bih_ref, bhh_ref,
                cf_ref, asn_ref, xw_ref):
    f32 = jnp.float32
    w1x = w1x_ref[...]

    xsum = jnp.zeros((_B, _F), f32)
    for j in range(_NCH):
        xc = x_ref[:, j * _NC:(j + 1) * _NC, :]
        xw_ref[:, j * _NC:(j + 1) * _NC, :] = _dot_nt(
            xc.reshape(_B * _NC, _F), w1x).reshape(_B, _NC, _H)
        xsum = xsum + jnp.sum(xc, axis=1)
    xm = xsum * (1.0 / _N)

    gc = _dot_nt(xm, wc_ref[...]) + bc_ref[...]

    avail = maskf_ref[:, 0, :] > 0.5
    h = jnp.zeros((_B, _H), f32)
    iota_n = jax.lax.broadcasted_iota(jnp.int32, (_B, _N), 1)
    b1 = b1_ref[...]
    w2 = w2_ref[...]
    wih = wih_ref[...]
    whh = whh_ref[...]
    bih = bih_ref[...]
    bhh = bhh_ref[...]

    def gru_step(hh, git):
        gh = _dot_nt(hh, whh) + bhh
        r = jax.nn.sigmoid(git[:, 0:_H] + gh[:, 0:_H])
        z = jax.nn.sigmoid(git[:, _H:2 * _H] + gh[:, _H:2 * _H])
        n = jnp.tanh(git[:, 2 * _H:] + r * gh[:, 2 * _H:])
        return (1.0 - z) * n + z * hh

    gis = []
    for c in range(_C):
        h_pre = h
        for t_ in range(c):
            h_pre = gru_step(h_pre, gis[t_])

        d = _dot_nt(gc, w1c_ref[...]) + b1
        lparts = []
        for j in range(_NCH):
            t = jnp.maximum(xw_ref[:, j * _NC:(j + 1) * _NC, :]
                            + d[:, None, :], 0.0)
            lg = _dot_nt(t.reshape(_B * _NC, _H), w2)
            lparts.append(lg.reshape(_B, _NC))
        logits = jnp.concatenate(lparts, axis=1)
        noisy = jnp.where(avail, logits + g_ref[:, c, :], -jnp.inf)

        m = jnp.max(noisy, axis=-1, keepdims=True)
        idx = jnp.min(jnp.where(noisy == m, iota_n, _N),
                      axis=-1, keepdims=True)
        has = jnp.any(avail, axis=-1, keepdims=True)
        sel = (iota_n == idx) & has
        onehot = sel.astype(f32)
        asn_ref[:, c, :] = onehot

        rows = []
        for b in range(_B):
            rows.append(x_ref[b, pl.ds(idx[b, 0], 1), :])
        emb = jnp.concatenate(rows, axis=0) * has.astype(f32)
        cf_ref[:, c, :] = emb

        gi = _dot_nt(emb, wih) + bih
        gis.append(gi)
        h = gru_step(h_pre, gi)
        gc = h
        avail = avail & (~sel)


def kernel(x, adj, mask, W1, b1, W2, b2, Wc, bc, W_ih, W_hh, b_ih, b_hh):
    del adj, b2
    f32 = jnp.float32

    noise_key = jax.random.key(123)
    g = jnp.stack([
        -jnp.log(-jnp.log(
            jax.random.uniform(jax.random.fold_in(noise_key, c), (_B, _N),
                               dtype=f32) + 1e-8) + 1e-8)
        for c in range(_C)
    ])
    g = g.transpose(1, 0, 2)

    maskf = mask.astype(f32).reshape(_B, 1, _N)
    w1x = W1[:, :_F]
    w1c = W1[:, _F:]

    cf, asn = pl.pallas_call(
        _fused_body,
        out_shape=[
            jax.ShapeDtypeStruct((_B, _C, _F), f32),
            jax.ShapeDtypeStruct((_B, _C, _N), f32),
        ],
        scratch_shapes=[pltpu.VMEM((_B, _N, _H), f32)],
    )(x, maskf, g, w1x, w1c, b1.reshape(1, _H), W2,
      Wc, bc.reshape(1, _H), W_ih, W_hh,
      b_ih.reshape(1, 3 * _H), b_hh.reshape(1, 3 * _H))

    asn = asn.transpose(0, 2, 1)
    cluster_adj = jnp.broadcast_to(
        (jnp.ones((_C, _C), f32) - jnp.eye(_C, dtype=f32))[None], (_B, _C, _C))
    return cf, cluster_adj, asn
